# baseline (speedup 1.0000x reference)
import functools

import jax
import jax.numpy as jnp
from jax import lax
from jax.experimental import pallas as pl
from jax.experimental.pallas import tpu as pltpu
from jax.experimental.pallas import tpu_sc as plsc

B = 16384
L = 200
NBINS = 1000
LANES = 16
NC, NS = 2, 16
NW = NC * NS
RW = B // NW
R = 32
NBLK = RW // R
HPAD = 1008
NCHUNK = HPAD // LANES
LCH = (L + LANES - 1) // LANES
TAIL = L - (LCH - 1) * LANES
BLKW = R * L

_mesh = plsc.VectorSubcoreMesh(
    core_axis_name="c", subcore_axis_name="s", num_cores=NC, num_subcores=NS
)


@functools.partial(
    pl.kernel,
    out_type=(
        jax.ShapeDtypeStruct((B * 48,), jnp.float32),
        jax.ShapeDtypeStruct((NW * 64,), jnp.float32),
    ),
    mesh=_mesh,
    compiler_params=pltpu.CompilerParams(needs_layout_passes=False),
    scratch_types=[
        pltpu.VMEM((2 * BLKW + LANES,), jnp.float32),
        pltpu.VMEM((2 * BLKW + LANES,), jnp.float32),
        pltpu.VMEM((2 * BLKW + LANES,), jnp.float32),
        pltpu.VMEM((2 * BLKW + LANES,), jnp.float32),
        pltpu.VMEM((HPAD,), jnp.float32),
        pltpu.VMEM((HPAD,), jnp.float32),
        pltpu.VMEM((HPAD,), jnp.float32),
        pltpu.VMEM((HPAD,), jnp.float32),
        pltpu.VMEM((2 * R * 48,), jnp.float32),
        pltpu.VMEM((64,), jnp.float32),
        pltpu.SemaphoreType.DMA((2,)),
        pltpu.SemaphoreType.DMA((2,)),
    ],
)
def _hybrid_loss_sc(pmz_h, pin_h, tmz_h, tin_h, rows_out, glob_out,
                    pmz_v, pin_v, tmz_v, tin_v, hp, ht, hp1, ht1, outs, gstage,
                    sem_in, sem_out):
    wid = lax.axis_index("s") * NC + lax.axis_index("c")
    base = wid * RW
    zeros16 = jnp.zeros((LANES,), jnp.float32)
    tail_mask = lax.iota(jnp.int32, LANES) < TAIL
    bufs = (pmz_v, pin_v, tmz_v, tin_v)
    hbms = (pmz_h, pin_h, tmz_h, tin_h)

    def start_in(blk, slot):
        src = (base + blk * R) * L
        dst = slot * BLKW
        for h, v in zip(hbms, bufs):
            pltpu.async_copy(h.at[pl.ds(src, BLKW)], v.at[pl.ds(dst, BLKW)],
                             sem_in.at[slot])

    def wait_in(slot):
        dst = slot * BLKW
        for h, v in zip(hbms, bufs):
            pltpu.make_async_copy(h.at[pl.ds(0, BLKW)], v.at[pl.ds(dst, BLKW)],
                                  sem_in.at[slot]).wait()

    def wait_out(slot):
        o = slot * R * 48
        pltpu.make_async_copy(rows_out.at[pl.ds(0, R * 48)],
                              outs.at[pl.ds(o, R * 48)], sem_out.at[slot]).wait()

    for j in range(NCHUNK):
        hp[pl.ds(j * LANES, LANES)] = zeros16
        ht[pl.ds(j * LANES, LANES)] = zeros16
        hp1[pl.ds(j * LANES, LANES)] = zeros16
        ht1[pl.ds(j * LANES, LANES)] = zeros16

    def scatter_pair(off0):
        off1b = off0 + L

        @plsc.parallel_loop(0, LCH, unroll=4)
        def _scat(c):
            off = off0 + c * LANES
            offb = off1b + c * LANES
            m = lax.iota(jnp.int32, LANES) < jnp.minimum(L - c * LANES, LANES)
            pmzi = pmz_v[pl.ds(off, LANES)].astype(jnp.int32)
            pbin = jnp.maximum(pmzi - 1, 0)
            plsc.addupdate_scatter(hp, [pbin], pin_v[pl.ds(off, LANES)], mask=m)
            qmzi = pmz_v[pl.ds(offb, LANES)].astype(jnp.int32)
            qbin = jnp.maximum(qmzi - 1, 0)
            plsc.addupdate_scatter(hp1, [qbin], pin_v[pl.ds(offb, LANES)], mask=m)
            tmzi = tmz_v[pl.ds(off, LANES)].astype(jnp.int32)
            tbin = jnp.maximum(tmzi - 1, 0)
            plsc.addupdate_scatter(ht, [tbin], tin_v[pl.ds(off, LANES)], mask=m)
            umzi = tmz_v[pl.ds(offb, LANES)].astype(jnp.int32)
            ubin = jnp.maximum(umzi - 1, 0)
            plsc.addupdate_scatter(ht1, [ubin], tin_v[pl.ds(offb, LANES)], mask=m)

    def sweep_pair():
        @plsc.parallel_loop(0, NCHUNK, carry=(zeros16,) * 10, unroll=2)
        def acc(j, c):
            dot0, na0, nb0, dot1, na1, nb1, n1, nt, n11, hs = c
            o = j * LANES
            p0 = hp[pl.ds(o, LANES)]
            t0 = ht[pl.ds(o, LANES)]
            p1 = hp1[pl.ds(o, LANES)]
            t1 = ht1[pl.ds(o, LANES)]
            hp[pl.ds(o, LANES)] = zeros16
            ht[pl.ds(o, LANES)] = zeros16
            hp1[pl.ds(o, LANES)] = zeros16
            ht1[pl.ds(o, LANES)] = zeros16
            dot0 = dot0 + p0 * t0
            na0 = na0 + p0 * p0
            nb0 = nb0 + t0 * t0
            dot1 = dot1 + p1 * t1
            na1 = na1 + p1 * p1
            nb1 = nb1 + t1 * t1
            pe0 = p0 > 0.0
            te0 = t0 > 0.0
            pe1 = p1 > 0.0
            te1 = t1 > 0.0
            pef0 = pe0.astype(jnp.float32)
            tef0 = te0.astype(jnp.float32)
            pef1 = pe1.astype(jnp.float32)
            tef1 = te1.astype(jnp.float32)
            n1 = n1 + (pef0 + pef1)
            nt = nt + (tef0 + tef1)
            n11 = n11 + (pef0 * tef0 + pef1 * tef1)
            d0 = p0 - t0
            ad0 = jnp.abs(d0)
            el0 = jnp.where(ad0 < 1.0, (0.5 * d0) * d0, ad0 - 0.5)
            d1 = p1 - t1
            ad1 = jnp.abs(d1)
            el1 = jnp.where(ad1 < 1.0, (0.5 * d1) * d1, ad1 - 0.5)
            hs = hs + (jnp.where(te0, el0, 0.0) + jnp.where(te1, el1, 0.0))
            return (dot0, na0, nb0, dot1, na1, nb1, n1, nt, n11, hs)
        return acc

    def blk_body(blk, gacc):
        slot = lax.rem(blk, 2)
        start_in(jnp.minimum(blk + 1, NBLK - 1), 1 - slot)
        wait_in(slot)

        @pl.when(blk >= 2)
        def _():
            wait_out(slot)

        in_off = slot * BLKW
        o_off = slot * R * 48

        def pair_body(rp, gacc):
            n1g, ntg, n11g, hsg = gacc
            scatter_pair(in_off + (2 * rp) * L)
            dot0, na0, nb0, dot1, na1, nb1, n1, nt, n11, hs = sweep_pair()
            o = o_off + (2 * rp) * 48
            outs[pl.ds(o, LANES)] = dot0
            outs[pl.ds(o + 16, LANES)] = na0
            outs[pl.ds(o + 32, LANES)] = nb0
            outs[pl.ds(o + 48, LANES)] = dot1
            outs[pl.ds(o + 64, LANES)] = na1
            outs[pl.ds(o + 80, LANES)] = nb1
            return (n1g + n1, ntg + nt, n11g + n11, hsg + hs)

        gacc = lax.fori_loop(0, R // 2, pair_body, gacc)
        pltpu.async_copy(outs.at[pl.ds(o_off, R * 48)],
                         rows_out.at[pl.ds((base + blk * R) * 48, R * 48)],
                         sem_out.at[slot])
        return gacc

    start_in(0, 0)
    z4 = (zeros16,) * 4
    n1, nt, n11, hs = lax.fori_loop(0, NBLK, blk_body, z4)
    wait_in(0)
    wait_out(0)
    wait_out(1)
    gstage[pl.ds(0, LANES)] = n1
    gstage[pl.ds(16, LANES)] = nt
    gstage[pl.ds(32, LANES)] = n11
    gstage[pl.ds(48, LANES)] = hs
    pltpu.sync_copy(gstage, glob_out.at[pl.ds(wid * 64, 64)])


def kernel(pred_mz, pred_intensity, true_mz, true_intensity):
    rows48, glob = _hybrid_loss_sc(
        pred_mz.reshape(-1), pred_intensity.reshape(-1),
        true_mz.reshape(-1), true_intensity.reshape(-1),
    )
    rows = rows48.reshape(B, 3, LANES).sum(-1)
    dot, na2, nb2 = rows[:, 0], rows[:, 1], rows[:, 2]
    na = jnp.maximum(jnp.sqrt(na2), 1e-8)
    nb = jnp.maximum(jnp.sqrt(nb2), 1e-8)
    loss_cosine = jnp.mean(1.0 - dot / (na * nb))

    g = glob.reshape(NW, 4, LANES).sum(axis=(0, 2))
    n1, ntc, n11, hs = g[0], g[1], g[2], g[3]
    n = jnp.float32(B * NBINS)
    c0 = jnp.log1p(jnp.exp(jnp.float32(0.0)))
    c1 = 1.0 + jnp.log1p(jnp.exp(jnp.float32(-1.0)))
    loss_peak = ((n - n1) * c0 + n1 * c1 - n11) / n
    loss_intensity = hs / jnp.maximum(ntc, 1.0)
    return 0.4 * loss_cosine + 0.3 * loss_peak + 0.2 * loss_intensity


# speedup vs baseline: 1.0069x; 1.0069x over previous
import functools

import jax
import jax.numpy as jnp
from jax import lax
from jax.experimental import pallas as pl
from jax.experimental.pallas import tpu as pltpu
from jax.experimental.pallas import tpu_sc as plsc

B = 16384
L = 200
NBINS = 1000
LANES = 16
NC, NS = 2, 16
NW = NC * NS
RW = B // NW
R = 16
NBLK = RW // R
HPAD = 1008
NCHUNK = HPAD // LANES
LCH = (L + LANES - 1) // LANES
TAIL = L - (LCH - 1) * LANES
BLKW = R * L

_mesh = plsc.VectorSubcoreMesh(
    core_axis_name="c", subcore_axis_name="s", num_cores=NC, num_subcores=NS
)


@functools.partial(
    pl.kernel,
    out_type=(
        jax.ShapeDtypeStruct((B * 48,), jnp.float32),
        jax.ShapeDtypeStruct((NW * 64,), jnp.float32),
    ),
    mesh=_mesh,
    compiler_params=pltpu.CompilerParams(needs_layout_passes=False),
    scratch_types=[
        pltpu.VMEM((2 * BLKW + LANES,), jnp.float32),
        pltpu.VMEM((2 * BLKW + LANES,), jnp.float32),
        pltpu.VMEM((2 * BLKW + LANES,), jnp.float32),
        pltpu.VMEM((2 * BLKW + LANES,), jnp.float32),
        pltpu.VMEM((HPAD,), jnp.float32),
        pltpu.VMEM((HPAD,), jnp.float32),
        pltpu.VMEM((HPAD,), jnp.float32),
        pltpu.VMEM((HPAD,), jnp.float32),
        pltpu.VMEM((2 * R * 48,), jnp.float32),
        pltpu.VMEM((64,), jnp.float32),
        pltpu.SemaphoreType.DMA((2,)),
        pltpu.SemaphoreType.DMA((2,)),
    ],
)
def _hybrid_loss_sc(pmz_h, pin_h, tmz_h, tin_h, rows_out, glob_out,
                    pmz_v, pin_v, tmz_v, tin_v, hp, ht, hp1, ht1, outs, gstage,
                    sem_in, sem_out):
    wid = lax.axis_index("s") * NC + lax.axis_index("c")
    base = wid * RW
    zeros16 = jnp.zeros((LANES,), jnp.float32)
    tail_mask = lax.iota(jnp.int32, LANES) < TAIL
    bufs = (pmz_v, pin_v, tmz_v, tin_v)
    hbms = (pmz_h, pin_h, tmz_h, tin_h)

    def start_in(blk, slot):
        src = (base + blk * R) * L
        dst = slot * BLKW
        for h, v in zip(hbms, bufs):
            pltpu.async_copy(h.at[pl.ds(src, BLKW)], v.at[pl.ds(dst, BLKW)],
                             sem_in.at[slot])

    def wait_in(slot):
        dst = slot * BLKW
        for h, v in zip(hbms, bufs):
            pltpu.make_async_copy(h.at[pl.ds(0, BLKW)], v.at[pl.ds(dst, BLKW)],
                                  sem_in.at[slot]).wait()

    def wait_out(slot):
        o = slot * R * 48
        pltpu.make_async_copy(rows_out.at[pl.ds(0, R * 48)],
                              outs.at[pl.ds(o, R * 48)], sem_out.at[slot]).wait()

    for j in range(NCHUNK):
        hp[pl.ds(j * LANES, LANES)] = zeros16
        ht[pl.ds(j * LANES, LANES)] = zeros16
        hp1[pl.ds(j * LANES, LANES)] = zeros16
        ht1[pl.ds(j * LANES, LANES)] = zeros16

    def scatter_pair(off0):
        off1b = off0 + L

        @plsc.parallel_loop(0, LCH, unroll=4)
        def _scat(c):
            off = off0 + c * LANES
            offb = off1b + c * LANES
            m = lax.iota(jnp.int32, LANES) < jnp.minimum(L - c * LANES, LANES)
            pmzi = pmz_v[pl.ds(off, LANES)].astype(jnp.int32)
            pbin = jnp.maximum(pmzi - 1, 0)
            plsc.addupdate_scatter(hp, [pbin], pin_v[pl.ds(off, LANES)], mask=m)
            qmzi = pmz_v[pl.ds(offb, LANES)].astype(jnp.int32)
            qbin = jnp.maximum(qmzi - 1, 0)
            plsc.addupdate_scatter(hp1, [qbin], pin_v[pl.ds(offb, LANES)], mask=m)
            tmzi = tmz_v[pl.ds(off, LANES)].astype(jnp.int32)
            tbin = jnp.maximum(tmzi - 1, 0)
            plsc.addupdate_scatter(ht, [tbin], tin_v[pl.ds(off, LANES)], mask=m)
            umzi = tmz_v[pl.ds(offb, LANES)].astype(jnp.int32)
            ubin = jnp.maximum(umzi - 1, 0)
            plsc.addupdate_scatter(ht1, [ubin], tin_v[pl.ds(offb, LANES)], mask=m)

    def sweep_pair():
        @plsc.parallel_loop(0, NCHUNK, carry=(zeros16,) * 10, unroll=3)
        def acc(j, c):
            dot0, na0, nb0, dot1, na1, nb1, n1, nt, n11, hs = c
            o = j * LANES
            p0 = hp[pl.ds(o, LANES)]
            t0 = ht[pl.ds(o, LANES)]
            p1 = hp1[pl.ds(o, LANES)]
            t1 = ht1[pl.ds(o, LANES)]
            hp[pl.ds(o, LANES)] = zeros16
            ht[pl.ds(o, LANES)] = zeros16
            hp1[pl.ds(o, LANES)] = zeros16
            ht1[pl.ds(o, LANES)] = zeros16
            dot0 = dot0 + p0 * t0
            na0 = na0 + p0 * p0
            nb0 = nb0 + t0 * t0
            dot1 = dot1 + p1 * t1
            na1 = na1 + p1 * p1
            nb1 = nb1 + t1 * t1
            pe0 = p0 > 0.0
            te0 = t0 > 0.0
            pe1 = p1 > 0.0
            te1 = t1 > 0.0
            pef0 = pe0.astype(jnp.float32)
            tef0 = te0.astype(jnp.float32)
            pef1 = pe1.astype(jnp.float32)
            tef1 = te1.astype(jnp.float32)
            n1 = n1 + (pef0 + pef1)
            nt = nt + (tef0 + tef1)
            n11 = n11 + (pef0 * tef0 + pef1 * tef1)
            d0 = p0 - t0
            ad0 = jnp.abs(d0)
            el0 = jnp.where(ad0 < 1.0, (0.5 * d0) * d0, ad0 - 0.5)
            d1 = p1 - t1
            ad1 = jnp.abs(d1)
            el1 = jnp.where(ad1 < 1.0, (0.5 * d1) * d1, ad1 - 0.5)
            hs = hs + (jnp.where(te0, el0, 0.0) + jnp.where(te1, el1, 0.0))
            return (dot0, na0, nb0, dot1, na1, nb1, n1, nt, n11, hs)
        return acc

    def blk_body(blk, gacc):
        slot = lax.rem(blk, 2)
        start_in(jnp.minimum(blk + 1, NBLK - 1), 1 - slot)
        wait_in(slot)

        @pl.when(blk >= 2)
        def _():
            wait_out(slot)

        in_off = slot * BLKW
        o_off = slot * R * 48

        def pair_body(rp, gacc):
            n1g, ntg, n11g, hsg = gacc
            scatter_pair(in_off + (2 * rp) * L)
            dot0, na0, nb0, dot1, na1, nb1, n1, nt, n11, hs = sweep_pair()
            o = o_off + (2 * rp) * 48
            outs[pl.ds(o, LANES)] = dot0
            outs[pl.ds(o + 16, LANES)] = na0
            outs[pl.ds(o + 32, LANES)] = nb0
            outs[pl.ds(o + 48, LANES)] = dot1
            outs[pl.ds(o + 64, LANES)] = na1
            outs[pl.ds(o + 80, LANES)] = nb1
            return (n1g + n1, ntg + nt, n11g + n11, hsg + hs)

        gacc = lax.fori_loop(0, R // 2, pair_body, gacc)
        pltpu.async_copy(outs.at[pl.ds(o_off, R * 48)],
                         rows_out.at[pl.ds((base + blk * R) * 48, R * 48)],
                         sem_out.at[slot])
        return gacc

    start_in(0, 0)
    z4 = (zeros16,) * 4
    n1, nt, n11, hs = lax.fori_loop(0, NBLK, blk_body, z4)
    wait_in(0)
    wait_out(0)
    wait_out(1)
    gstage[pl.ds(0, LANES)] = n1
    gstage[pl.ds(16, LANES)] = nt
    gstage[pl.ds(32, LANES)] = n11
    gstage[pl.ds(48, LANES)] = hs
    pltpu.sync_copy(gstage, glob_out.at[pl.ds(wid * 64, 64)])


def kernel(pred_mz, pred_intensity, true_mz, true_intensity):
    rows48, glob = _hybrid_loss_sc(
        pred_mz.reshape(-1), pred_intensity.reshape(-1),
        true_mz.reshape(-1), true_intensity.reshape(-1),
    )
    rows = rows48.reshape(B, 3, LANES).sum(-1)
    dot, na2, nb2 = rows[:, 0], rows[:, 1], rows[:, 2]
    na = jnp.maximum(jnp.sqrt(na2), 1e-8)
    nb = jnp.maximum(jnp.sqrt(nb2), 1e-8)
    loss_cosine = jnp.mean(1.0 - dot / (na * nb))

    g = glob.reshape(NW, 4, LANES).sum(axis=(0, 2))
    n1, ntc, n11, hs = g[0], g[1], g[2], g[3]
    n = jnp.float32(B * NBINS)
    c0 = jnp.log1p(jnp.exp(jnp.float32(0.0)))
    c1 = 1.0 + jnp.log1p(jnp.exp(jnp.float32(-1.0)))
    loss_peak = ((n - n1) * c0 + n1 * c1 - n11) / n
    loss_intensity = hs / jnp.maximum(ntc, 1.0)
    return 0.4 * loss_cosine + 0.3 * loss_peak + 0.2 * loss_intensity


# speedup vs baseline: 1.0192x; 1.0121x over previous
import functools

import jax
import jax.numpy as jnp
from jax import lax
from jax.experimental import pallas as pl
from jax.experimental.pallas import tpu as pltpu
from jax.experimental.pallas import tpu_sc as plsc

B = 16384
L = 200
NBINS = 1000
LANES = 16
NC, NS = 2, 16
NW = NC * NS
RW = B // NW
R = 16
NBLK = RW // R
HPAD = 1008
NCHUNK = HPAD // LANES
LCH = (L + LANES - 1) // LANES
TAIL = L - (LCH - 1) * LANES
BLKW = R * L

_mesh = plsc.VectorSubcoreMesh(
    core_axis_name="c", subcore_axis_name="s", num_cores=NC, num_subcores=NS
)


@functools.partial(
    pl.kernel,
    out_type=(
        jax.ShapeDtypeStruct((B * 48,), jnp.float32),
        jax.ShapeDtypeStruct((NW * 64,), jnp.float32),
    ),
    mesh=_mesh,
    compiler_params=pltpu.CompilerParams(needs_layout_passes=False),
    scratch_types=[
        pltpu.VMEM((2 * BLKW + LANES,), jnp.float32),
        pltpu.VMEM((2 * BLKW + LANES,), jnp.float32),
        pltpu.VMEM((2 * BLKW + LANES,), jnp.float32),
        pltpu.VMEM((2 * BLKW + LANES,), jnp.float32),
        pltpu.VMEM((HPAD,), jnp.float32),
        pltpu.VMEM((HPAD,), jnp.float32),
        pltpu.VMEM((HPAD,), jnp.float32),
        pltpu.VMEM((HPAD,), jnp.float32),
        pltpu.VMEM((2 * R * 48,), jnp.float32),
        pltpu.VMEM((64,), jnp.float32),
        pltpu.SemaphoreType.DMA((2,)),
        pltpu.SemaphoreType.DMA((2,)),
    ],
)
def _hybrid_loss_sc(pmz_h, pin_h, tmz_h, tin_h, rows_out, glob_out,
                    pmz_v, pin_v, tmz_v, tin_v, hp, ht, hp1, ht1, outs, gstage,
                    sem_in, sem_out):
    wid = lax.axis_index("s") * NC + lax.axis_index("c")
    base = wid * RW
    zeros16 = jnp.zeros((LANES,), jnp.float32)
    tail_mask = lax.iota(jnp.int32, LANES) < TAIL
    bufs = (pmz_v, pin_v, tmz_v, tin_v)
    hbms = (pmz_h, pin_h, tmz_h, tin_h)

    def start_in(blk, slot):
        src = (base + blk * R) * L
        dst = slot * BLKW
        for h, v in zip(hbms, bufs):
            pltpu.async_copy(h.at[pl.ds(src, BLKW)], v.at[pl.ds(dst, BLKW)],
                             sem_in.at[slot])

    def wait_in(slot):
        dst = slot * BLKW
        for h, v in zip(hbms, bufs):
            pltpu.make_async_copy(h.at[pl.ds(0, BLKW)], v.at[pl.ds(dst, BLKW)],
                                  sem_in.at[slot]).wait()

    def wait_out(slot):
        o = slot * R * 48
        pltpu.make_async_copy(rows_out.at[pl.ds(0, R * 48)],
                              outs.at[pl.ds(o, R * 48)], sem_out.at[slot]).wait()

    for j in range(NCHUNK):
        hp[pl.ds(j * LANES, LANES)] = zeros16
        ht[pl.ds(j * LANES, LANES)] = zeros16
        hp1[pl.ds(j * LANES, LANES)] = zeros16
        ht1[pl.ds(j * LANES, LANES)] = zeros16

    def scatter_pair(off0):
        off1b = off0 + L

        @plsc.parallel_loop(0, LCH, unroll=4)
        def _scat(c):
            off = off0 + c * LANES
            offb = off1b + c * LANES
            m = lax.iota(jnp.int32, LANES) < jnp.minimum(L - c * LANES, LANES)
            pmzi = pmz_v[pl.ds(off, LANES)].astype(jnp.int32)
            pbin = jnp.maximum(pmzi - 1, 0)
            plsc.addupdate_scatter(hp, [pbin], pin_v[pl.ds(off, LANES)], mask=m)
            qmzi = pmz_v[pl.ds(offb, LANES)].astype(jnp.int32)
            qbin = jnp.maximum(qmzi - 1, 0)
            plsc.addupdate_scatter(hp1, [qbin], pin_v[pl.ds(offb, LANES)], mask=m)
            tmzi = tmz_v[pl.ds(off, LANES)].astype(jnp.int32)
            tbin = jnp.maximum(tmzi - 1, 0)
            plsc.addupdate_scatter(ht, [tbin], tin_v[pl.ds(off, LANES)], mask=m)
            umzi = tmz_v[pl.ds(offb, LANES)].astype(jnp.int32)
            ubin = jnp.maximum(umzi - 1, 0)
            plsc.addupdate_scatter(ht1, [ubin], tin_v[pl.ds(offb, LANES)], mask=m)

    def sweep_pair():
        @plsc.parallel_loop(0, NCHUNK, carry=(zeros16,) * 10, unroll=3)
        def acc(j, c):
            dot0, na0, nb0, dot1, na1, nb1, n1, nt, n11, hs = c
            o = j * LANES
            p0 = hp[pl.ds(o, LANES)]
            t0 = ht[pl.ds(o, LANES)]
            p1 = hp1[pl.ds(o, LANES)]
            t1 = ht1[pl.ds(o, LANES)]
            hp[pl.ds(o, LANES)] = zeros16
            ht[pl.ds(o, LANES)] = zeros16
            hp1[pl.ds(o, LANES)] = zeros16
            ht1[pl.ds(o, LANES)] = zeros16
            dot0 = dot0 + p0 * t0
            na0 = na0 + p0 * p0
            nb0 = nb0 + t0 * t0
            dot1 = dot1 + p1 * t1
            na1 = na1 + p1 * p1
            nb1 = nb1 + t1 * t1
            pe0 = p0 > 0.0
            te0 = t0 > 0.0
            pe1 = p1 > 0.0
            te1 = t1 > 0.0
            pef0 = pe0.astype(jnp.float32)
            tef0 = te0.astype(jnp.float32)
            pef1 = pe1.astype(jnp.float32)
            tef1 = te1.astype(jnp.float32)
            n1 = n1 + (pef0 + pef1)
            nt = nt + (tef0 + tef1)
            n11 = n11 + (pef0 * tef0 + pef1 * tef1)
            ad0 = jnp.abs(p0 - t0)
            c0 = jnp.minimum(ad0, 1.0)
            el0 = c0 * (ad0 - 0.5 * c0)
            ad1 = jnp.abs(p1 - t1)
            c1 = jnp.minimum(ad1, 1.0)
            el1 = c1 * (ad1 - 0.5 * c1)
            hs = hs + (jnp.where(te0, el0, 0.0) + jnp.where(te1, el1, 0.0))
            return (dot0, na0, nb0, dot1, na1, nb1, n1, nt, n11, hs)
        return acc

    def blk_body(blk, gacc):
        slot = lax.rem(blk, 2)
        start_in(jnp.minimum(blk + 1, NBLK - 1), 1 - slot)
        wait_in(slot)

        @pl.when(blk >= 2)
        def _():
            wait_out(slot)

        in_off = slot * BLKW
        o_off = slot * R * 48

        def pair_body(rp, gacc):
            n1g, ntg, n11g, hsg = gacc
            scatter_pair(in_off + (2 * rp) * L)
            dot0, na0, nb0, dot1, na1, nb1, n1, nt, n11, hs = sweep_pair()
            o = o_off + (2 * rp) * 48
            outs[pl.ds(o, LANES)] = dot0
            outs[pl.ds(o + 16, LANES)] = na0
            outs[pl.ds(o + 32, LANES)] = nb0
            outs[pl.ds(o + 48, LANES)] = dot1
            outs[pl.ds(o + 64, LANES)] = na1
            outs[pl.ds(o + 80, LANES)] = nb1
            return (n1g + n1, ntg + nt, n11g + n11, hsg + hs)

        gacc = lax.fori_loop(0, R // 2, pair_body, gacc)
        pltpu.async_copy(outs.at[pl.ds(o_off, R * 48)],
                         rows_out.at[pl.ds((base + blk * R) * 48, R * 48)],
                         sem_out.at[slot])
        return gacc

    start_in(0, 0)
    z4 = (zeros16,) * 4
    n1, nt, n11, hs = lax.fori_loop(0, NBLK, blk_body, z4)
    wait_in(0)
    wait_out(0)
    wait_out(1)
    gstage[pl.ds(0, LANES)] = n1
    gstage[pl.ds(16, LANES)] = nt
    gstage[pl.ds(32, LANES)] = n11
    gstage[pl.ds(48, LANES)] = hs
    pltpu.sync_copy(gstage, glob_out.at[pl.ds(wid * 64, 64)])


def kernel(pred_mz, pred_intensity, true_mz, true_intensity):
    rows48, glob = _hybrid_loss_sc(
        pred_mz.reshape(-1), pred_intensity.reshape(-1),
        true_mz.reshape(-1), true_intensity.reshape(-1),
    )
    rows = rows48.reshape(B, 3, LANES).sum(-1)
    dot, na2, nb2 = rows[:, 0], rows[:, 1], rows[:, 2]
    na = jnp.maximum(jnp.sqrt(na2), 1e-8)
    nb = jnp.maximum(jnp.sqrt(nb2), 1e-8)
    loss_cosine = jnp.mean(1.0 - dot / (na * nb))

    g = glob.reshape(NW, 4, LANES).sum(axis=(0, 2))
    n1, ntc, n11, hs = g[0], g[1], g[2], g[3]
    n = jnp.float32(B * NBINS)
    c0 = jnp.log1p(jnp.exp(jnp.float32(0.0)))
    c1 = 1.0 + jnp.log1p(jnp.exp(jnp.float32(-1.0)))
    loss_peak = ((n - n1) * c0 + n1 * c1 - n11) / n
    loss_intensity = hs / jnp.maximum(ntc, 1.0)
    return 0.4 * loss_cosine + 0.3 * loss_peak + 0.2 * loss_intensity


# speedup vs baseline: 1.0193x; 1.0001x over previous
import functools

import jax
import jax.numpy as jnp
from jax import lax
from jax.experimental import pallas as pl
from jax.experimental.pallas import tpu as pltpu
from jax.experimental.pallas import tpu_sc as plsc

B = 16384
L = 200
NBINS = 1000
LANES = 16
NC, NS = 2, 16
NW = NC * NS
RW = B // NW
R = 16
NBLK = RW // R
HPAD = 1008
NCHUNK = HPAD // LANES
LCH = (L + LANES - 1) // LANES
TAIL = L - (LCH - 1) * LANES
BLKW = R * L

_mesh = plsc.VectorSubcoreMesh(
    core_axis_name="c", subcore_axis_name="s", num_cores=NC, num_subcores=NS
)


@functools.partial(
    pl.kernel,
    out_type=(
        jax.ShapeDtypeStruct((B * 48,), jnp.float32),
        jax.ShapeDtypeStruct((NW * 64,), jnp.float32),
    ),
    mesh=_mesh,
    compiler_params=pltpu.CompilerParams(needs_layout_passes=False),
    scratch_types=[
        pltpu.VMEM((2 * BLKW + LANES,), jnp.float32),
        pltpu.VMEM((2 * BLKW + LANES,), jnp.float32),
        pltpu.VMEM((2 * BLKW + LANES,), jnp.float32),
        pltpu.VMEM((2 * BLKW + LANES,), jnp.float32),
        pltpu.VMEM((HPAD,), jnp.float32),
        pltpu.VMEM((HPAD,), jnp.float32),
        pltpu.VMEM((HPAD,), jnp.float32),
        pltpu.VMEM((HPAD,), jnp.float32),
        pltpu.VMEM((2 * R * 48,), jnp.float32),
        pltpu.VMEM((64,), jnp.float32),
        pltpu.SemaphoreType.DMA((2,)),
        pltpu.SemaphoreType.DMA((2,)),
    ],
)
def _hybrid_loss_sc(pmz_h, pin_h, tmz_h, tin_h, rows_out, glob_out,
                    pmz_v, pin_v, tmz_v, tin_v, hp, ht, hp1, ht1, outs, gstage,
                    sem_in, sem_out):
    wid = lax.axis_index("s") * NC + lax.axis_index("c")
    base = wid * RW
    zeros16 = jnp.zeros((LANES,), jnp.float32)
    tail_mask = lax.iota(jnp.int32, LANES) < TAIL
    bufs = (pmz_v, pin_v, tmz_v, tin_v)
    hbms = (pmz_h, pin_h, tmz_h, tin_h)

    def start_in(blk, slot):
        src = (base + blk * R) * L
        dst = slot * BLKW
        for h, v in zip(hbms, bufs):
            pltpu.async_copy(h.at[pl.ds(src, BLKW)], v.at[pl.ds(dst, BLKW)],
                             sem_in.at[slot])

    def wait_in(slot):
        dst = slot * BLKW
        for h, v in zip(hbms, bufs):
            pltpu.make_async_copy(h.at[pl.ds(0, BLKW)], v.at[pl.ds(dst, BLKW)],
                                  sem_in.at[slot]).wait()

    def wait_out(slot):
        o = slot * R * 48
        pltpu.make_async_copy(rows_out.at[pl.ds(0, R * 48)],
                              outs.at[pl.ds(o, R * 48)], sem_out.at[slot]).wait()

    for j in range(NCHUNK):
        hp[pl.ds(j * LANES, LANES)] = zeros16
        ht[pl.ds(j * LANES, LANES)] = zeros16
        hp1[pl.ds(j * LANES, LANES)] = zeros16
        ht1[pl.ds(j * LANES, LANES)] = zeros16

    def scatter_pair(off0):
        off1b = off0 + L

        def scat_chunk(off, offb, m):
            pmzi = pmz_v[pl.ds(off, LANES)].astype(jnp.int32)
            pbin = jnp.maximum(pmzi - 1, 0)
            plsc.addupdate_scatter(hp, [pbin], pin_v[pl.ds(off, LANES)], mask=m)
            qmzi = pmz_v[pl.ds(offb, LANES)].astype(jnp.int32)
            qbin = jnp.maximum(qmzi - 1, 0)
            plsc.addupdate_scatter(hp1, [qbin], pin_v[pl.ds(offb, LANES)], mask=m)
            tmzi = tmz_v[pl.ds(off, LANES)].astype(jnp.int32)
            tbin = jnp.maximum(tmzi - 1, 0)
            plsc.addupdate_scatter(ht, [tbin], tin_v[pl.ds(off, LANES)], mask=m)
            umzi = tmz_v[pl.ds(offb, LANES)].astype(jnp.int32)
            ubin = jnp.maximum(umzi - 1, 0)
            plsc.addupdate_scatter(ht1, [ubin], tin_v[pl.ds(offb, LANES)], mask=m)

        @plsc.parallel_loop(0, LCH - 1, unroll=4)
        def _scat(c):
            o = c * LANES
            scat_chunk(off0 + o, off1b + o, None)

        o = (LCH - 1) * LANES
        scat_chunk(off0 + o, off1b + o, tail_mask)

    def sweep_pair():
        @plsc.parallel_loop(0, NCHUNK, carry=(zeros16,) * 10, unroll=3)
        def acc(j, c):
            dot0, na0, nb0, dot1, na1, nb1, n1, nt, n11, hs = c
            o = j * LANES
            p0 = hp[pl.ds(o, LANES)]
            t0 = ht[pl.ds(o, LANES)]
            p1 = hp1[pl.ds(o, LANES)]
            t1 = ht1[pl.ds(o, LANES)]
            hp[pl.ds(o, LANES)] = zeros16
            ht[pl.ds(o, LANES)] = zeros16
            hp1[pl.ds(o, LANES)] = zeros16
            ht1[pl.ds(o, LANES)] = zeros16
            dot0 = dot0 + p0 * t0
            na0 = na0 + p0 * p0
            nb0 = nb0 + t0 * t0
            dot1 = dot1 + p1 * t1
            na1 = na1 + p1 * p1
            nb1 = nb1 + t1 * t1
            pe0 = p0 > 0.0
            te0 = t0 > 0.0
            pe1 = p1 > 0.0
            te1 = t1 > 0.0
            pef0 = pe0.astype(jnp.float32)
            tef0 = te0.astype(jnp.float32)
            pef1 = pe1.astype(jnp.float32)
            tef1 = te1.astype(jnp.float32)
            n1 = n1 + (pef0 + pef1)
            nt = nt + (tef0 + tef1)
            n11 = n11 + (pef0 * tef0 + pef1 * tef1)
            ad0 = jnp.abs(p0 - t0)
            c0 = jnp.minimum(ad0, 1.0)
            el0 = c0 * (ad0 - 0.5 * c0)
            ad1 = jnp.abs(p1 - t1)
            c1 = jnp.minimum(ad1, 1.0)
            el1 = c1 * (ad1 - 0.5 * c1)
            hs = hs + (jnp.where(te0, el0, 0.0) + jnp.where(te1, el1, 0.0))
            return (dot0, na0, nb0, dot1, na1, nb1, n1, nt, n11, hs)
        return acc

    def blk_body(blk, gacc):
        slot = lax.rem(blk, 2)
        start_in(jnp.minimum(blk + 1, NBLK - 1), 1 - slot)
        wait_in(slot)

        @pl.when(blk >= 2)
        def _():
            wait_out(slot)

        in_off = slot * BLKW
        o_off = slot * R * 48

        def pair_body(rp, gacc):
            n1g, ntg, n11g, hsg = gacc
            scatter_pair(in_off + (2 * rp) * L)
            dot0, na0, nb0, dot1, na1, nb1, n1, nt, n11, hs = sweep_pair()
            o = o_off + (2 * rp) * 48
            outs[pl.ds(o, LANES)] = dot0
            outs[pl.ds(o + 16, LANES)] = na0
            outs[pl.ds(o + 32, LANES)] = nb0
            outs[pl.ds(o + 48, LANES)] = dot1
            outs[pl.ds(o + 64, LANES)] = na1
            outs[pl.ds(o + 80, LANES)] = nb1
            return (n1g + n1, ntg + nt, n11g + n11, hsg + hs)

        gacc = lax.fori_loop(0, R // 2, pair_body, gacc)
        pltpu.async_copy(outs.at[pl.ds(o_off, R * 48)],
                         rows_out.at[pl.ds((base + blk * R) * 48, R * 48)],
                         sem_out.at[slot])
        return gacc

    start_in(0, 0)
    z4 = (zeros16,) * 4
    n1, nt, n11, hs = lax.fori_loop(0, NBLK, blk_body, z4)
    wait_in(0)
    wait_out(0)
    wait_out(1)
    gstage[pl.ds(0, LANES)] = n1
    gstage[pl.ds(16, LANES)] = nt
    gstage[pl.ds(32, LANES)] = n11
    gstage[pl.ds(48, LANES)] = hs
    pltpu.sync_copy(gstage, glob_out.at[pl.ds(wid * 64, 64)])


def kernel(pred_mz, pred_intensity, true_mz, true_intensity):
    rows48, glob = _hybrid_loss_sc(
        pred_mz.reshape(-1), pred_intensity.reshape(-1),
        true_mz.reshape(-1), true_intensity.reshape(-1),
    )
    rows = rows48.reshape(B, 3, LANES).sum(-1)
    dot, na2, nb2 = rows[:, 0], rows[:, 1], rows[:, 2]
    na = jnp.maximum(jnp.sqrt(na2), 1e-8)
    nb = jnp.maximum(jnp.sqrt(nb2), 1e-8)
    loss_cosine = jnp.mean(1.0 - dot / (na * nb))

    g = glob.reshape(NW, 4, LANES).sum(axis=(0, 2))
    n1, ntc, n11, hs = g[0], g[1], g[2], g[3]
    n = jnp.float32(B * NBINS)
    c0 = jnp.log1p(jnp.exp(jnp.float32(0.0)))
    c1 = 1.0 + jnp.log1p(jnp.exp(jnp.float32(-1.0)))
    loss_peak = ((n - n1) * c0 + n1 * c1 - n11) / n
    loss_intensity = hs / jnp.maximum(ntc, 1.0)
    return 0.4 * loss_cosine + 0.3 * loss_peak + 0.2 * loss_intensity


# speedup vs baseline: 1.0199x; 1.0006x over previous
import functools

import jax
import jax.numpy as jnp
from jax import lax
from jax.experimental import pallas as pl
from jax.experimental.pallas import tpu as pltpu
from jax.experimental.pallas import tpu_sc as plsc

B = 16384
L = 200
NBINS = 1000
LANES = 16
NC, NS = 2, 16
NW = NC * NS
RW = B // NW
R = 16
NBLK = RW // R
HPAD = 1008
NCHUNK = HPAD // LANES
LCH = (L + LANES - 1) // LANES
TAIL = L - (LCH - 1) * LANES
BLKW = R * L

_mesh = plsc.VectorSubcoreMesh(
    core_axis_name="c", subcore_axis_name="s", num_cores=NC, num_subcores=NS
)


@functools.partial(
    pl.kernel,
    out_type=(
        jax.ShapeDtypeStruct((B * 48,), jnp.float32),
        jax.ShapeDtypeStruct((NW * 64,), jnp.float32),
    ),
    mesh=_mesh,
    compiler_params=pltpu.CompilerParams(needs_layout_passes=False),
    scratch_types=[
        pltpu.VMEM((2 * BLKW + LANES,), jnp.float32),
        pltpu.VMEM((2 * BLKW + LANES,), jnp.float32),
        pltpu.VMEM((2 * BLKW + LANES,), jnp.float32),
        pltpu.VMEM((2 * BLKW + LANES,), jnp.float32),
        pltpu.VMEM((HPAD,), jnp.float32),
        pltpu.VMEM((HPAD,), jnp.float32),
        pltpu.VMEM((HPAD,), jnp.float32),
        pltpu.VMEM((HPAD,), jnp.float32),
        pltpu.VMEM((2 * R * 48,), jnp.float32),
        pltpu.VMEM((64,), jnp.float32),
        pltpu.SemaphoreType.DMA((2,)),
        pltpu.SemaphoreType.DMA((2,)),
    ],
)
def _hybrid_loss_sc(pmz_h, pin_h, tmz_h, tin_h, rows_out, glob_out,
                    pmz_v, pin_v, tmz_v, tin_v, hp, ht, hp1, ht1, outs, gstage,
                    sem_in, sem_out):
    wid = lax.axis_index("s") * NC + lax.axis_index("c")
    base = wid * RW
    zeros16 = jnp.zeros((LANES,), jnp.float32)
    tail_mask = lax.iota(jnp.int32, LANES) < TAIL
    bufs = (pmz_v, pin_v, tmz_v, tin_v)
    hbms = (pmz_h, pin_h, tmz_h, tin_h)

    def start_in(blk, slot):
        src = (base + blk * R) * L
        dst = slot * BLKW
        for h, v in zip(hbms, bufs):
            pltpu.async_copy(h.at[pl.ds(src, BLKW)], v.at[pl.ds(dst, BLKW)],
                             sem_in.at[slot])

    def wait_in(slot):
        dst = slot * BLKW
        for h, v in zip(hbms, bufs):
            pltpu.make_async_copy(h.at[pl.ds(0, BLKW)], v.at[pl.ds(dst, BLKW)],
                                  sem_in.at[slot]).wait()

    def wait_out(slot):
        o = slot * R * 48
        pltpu.make_async_copy(rows_out.at[pl.ds(0, R * 48)],
                              outs.at[pl.ds(o, R * 48)], sem_out.at[slot]).wait()

    for j in range(NCHUNK):
        hp[pl.ds(j * LANES, LANES)] = zeros16
        ht[pl.ds(j * LANES, LANES)] = zeros16
        hp1[pl.ds(j * LANES, LANES)] = zeros16
        ht1[pl.ds(j * LANES, LANES)] = zeros16

    def scatter_pair(off0):
        off1b = off0 + L

        @plsc.parallel_loop(0, LCH, unroll=4)
        def _scat(c):
            off = off0 + c * LANES
            offb = off1b + c * LANES
            m = lax.iota(jnp.int32, LANES) < jnp.minimum(L - c * LANES, LANES)
            pmzi = pmz_v[pl.ds(off, LANES)].astype(jnp.int32)
            pbin = jnp.maximum(pmzi - 1, 0)
            plsc.addupdate_scatter(hp, [pbin], pin_v[pl.ds(off, LANES)], mask=m)
            qmzi = pmz_v[pl.ds(offb, LANES)].astype(jnp.int32)
            qbin = jnp.maximum(qmzi - 1, 0)
            plsc.addupdate_scatter(hp1, [qbin], pin_v[pl.ds(offb, LANES)], mask=m)
            tmzi = tmz_v[pl.ds(off, LANES)].astype(jnp.int32)
            tbin = jnp.maximum(tmzi - 1, 0)
            plsc.addupdate_scatter(ht, [tbin], tin_v[pl.ds(off, LANES)], mask=m)
            umzi = tmz_v[pl.ds(offb, LANES)].astype(jnp.int32)
            ubin = jnp.maximum(umzi - 1, 0)
            plsc.addupdate_scatter(ht1, [ubin], tin_v[pl.ds(offb, LANES)], mask=m)

    def sweep_pair():
        @plsc.parallel_loop(0, NCHUNK, carry=(zeros16,) * 10, unroll=3)
        def acc(j, c):
            dot0, na0, nb0, dot1, na1, nb1, n1, nt, n11, hs = c
            o = j * LANES
            p0 = hp[pl.ds(o, LANES)]
            t0 = ht[pl.ds(o, LANES)]
            p1 = hp1[pl.ds(o, LANES)]
            t1 = ht1[pl.ds(o, LANES)]
            hp[pl.ds(o, LANES)] = zeros16
            ht[pl.ds(o, LANES)] = zeros16
            hp1[pl.ds(o, LANES)] = zeros16
            ht1[pl.ds(o, LANES)] = zeros16
            dot0 = dot0 + p0 * t0
            na0 = na0 + p0 * p0
            nb0 = nb0 + t0 * t0
            dot1 = dot1 + p1 * t1
            na1 = na1 + p1 * p1
            nb1 = nb1 + t1 * t1
            pe0 = p0 > 0.0
            te0 = t0 > 0.0
            pe1 = p1 > 0.0
            te1 = t1 > 0.0
            pef0 = pe0.astype(jnp.float32)
            tef0 = te0.astype(jnp.float32)
            pef1 = pe1.astype(jnp.float32)
            tef1 = te1.astype(jnp.float32)
            n1 = n1 + (pef0 + pef1)
            nt = nt + (tef0 + tef1)
            n11 = n11 + (pef0 * tef0 + pef1 * tef1)
            ad0 = jnp.abs(p0 - t0)
            c0 = jnp.minimum(ad0, 1.0)
            el0 = c0 * (ad0 - 0.5 * c0)
            ad1 = jnp.abs(p1 - t1)
            c1 = jnp.minimum(ad1, 1.0)
            el1 = c1 * (ad1 - 0.5 * c1)
            hs = hs + (jnp.where(te0, el0, 0.0) + jnp.where(te1, el1, 0.0))
            return (dot0, na0, nb0, dot1, na1, nb1, n1, nt, n11, hs)
        return acc

    def blk_body(blk, gacc):
        slot = lax.rem(blk, 2)
        start_in(jnp.minimum(blk + 1, NBLK - 1), 1 - slot)
        wait_in(slot)

        @pl.when(blk >= 2)
        def _():
            wait_out(slot)

        in_off = slot * BLKW
        o_off = slot * R * 48

        def pair_body(rp, gacc):
            n1g, ntg, n11g, hsg = gacc
            scatter_pair(in_off + (2 * rp) * L)
            dot0, na0, nb0, dot1, na1, nb1, n1, nt, n11, hs = sweep_pair()
            o = o_off + (2 * rp) * 48
            outs[pl.ds(o, LANES)] = dot0
            outs[pl.ds(o + 16, LANES)] = na0
            outs[pl.ds(o + 32, LANES)] = nb0
            outs[pl.ds(o + 48, LANES)] = dot1
            outs[pl.ds(o + 64, LANES)] = na1
            outs[pl.ds(o + 80, LANES)] = nb1
            return (n1g + n1, ntg + nt, n11g + n11, hsg + hs)

        gacc = lax.fori_loop(0, R // 2, pair_body, gacc)
        pltpu.async_copy(outs.at[pl.ds(o_off, R * 48)],
                         rows_out.at[pl.ds((base + blk * R) * 48, R * 48)],
                         sem_out.at[slot])
        return gacc

    start_in(0, 0)
    z4 = (zeros16,) * 4
    n1, nt, n11, hs = lax.fori_loop(0, NBLK, blk_body, z4)
    wait_in(0)
    wait_out(0)
    wait_out(1)
    gstage[pl.ds(0, LANES)] = n1
    gstage[pl.ds(16, LANES)] = nt
    gstage[pl.ds(32, LANES)] = n11
    gstage[pl.ds(48, LANES)] = hs
    pltpu.sync_copy(gstage, glob_out.at[pl.ds(wid * 64, 64)])


def kernel(pred_mz, pred_intensity, true_mz, true_intensity):
    rows48, glob = _hybrid_loss_sc(
        pred_mz.reshape(-1), pred_intensity.reshape(-1),
        true_mz.reshape(-1), true_intensity.reshape(-1),
    )
    rows = rows48.reshape(B, 3, LANES).sum(-1)
    dot, na2, nb2 = rows[:, 0], rows[:, 1], rows[:, 2]
    na = jnp.maximum(jnp.sqrt(na2), 1e-8)
    nb = jnp.maximum(jnp.sqrt(nb2), 1e-8)
    loss_cosine = jnp.mean(1.0 - dot / (na * nb))

    g = glob.reshape(NW, 4, LANES).sum(axis=(0, 2))
    n1, ntc, n11, hs = g[0], g[1], g[2], g[3]
    n = jnp.float32(B * NBINS)
    c0 = jnp.log1p(jnp.exp(jnp.float32(0.0)))
    c1 = 1.0 + jnp.log1p(jnp.exp(jnp.float32(-1.0)))
    loss_peak = ((n - n1) * c0 + n1 * c1 - n11) / n
    loss_intensity = hs / jnp.maximum(ntc, 1.0)
    return 0.4 * loss_cosine + 0.3 * loss_peak + 0.2 * loss_intensity
